# baseline (speedup 1.0000x reference)
import math

import jax
import jax.numpy as jnp
from jax.experimental import pallas as pl
from jax.experimental.pallas import tpu as pltpu

H = 128
F = 128
L = 6
NG = 50
NGP = 64
CUTOFF = 10.0
N = 4096
B = 64
TB = 64
NB = N // TB
GRID = NB * NB
SI = 64
ZPAD = 128
HT = TB // 2
P2H = TB * TB // 2
LOG2 = math.log(2.0)
DELTA = CUTOFF / (NG - 1)
COEFF = -0.5 / DELTA ** 2


def _ssp(x):
    return jax.nn.softplus(x) - LOG2


def _embed_kernel(z_r, emb_r, out_r):
    zcol = z_r[...]
    oh = (zcol == jax.lax.broadcasted_iota(jnp.int32, (TB, ZPAD), 1))
    out_r[...] = jnp.dot(oh.astype(jnp.float32), emb_r[...],
                         preferred_element_type=jnp.float32)


def _pair_kernel(ti_r, tj_r, cnt_r, posj_r, posit_r, bj_r, bi_r, out_r):
    p = pl.program_id(0)
    ti = ti_r[p]
    tj = tj_r[p]
    pj = posj_r[...]
    pit = posit_r[0]
    d2t = (pj[:, 0:1] - pit[0:1, :]) ** 2
    d2t = d2t + (pj[:, 1:2] - pit[1:2, :]) ** 2
    d2t = d2t + (pj[:, 2:3] - pit[2:3, :]) ** 2
    dt = jnp.sqrt(d2t)
    gj = tj * TB + jax.lax.broadcasted_iota(jnp.int32, (TB, TB), 0)
    gi = ti * TB + jax.lax.broadcasted_iota(jnp.int32, (TB, TB), 1)
    mt = (bj_r[...] == bi_r[0]) & (gj != gi) & (d2t < CUTOFF ** 2)
    cwt = 0.5 * (jnp.cos(dt * (jnp.pi / CUTOFF)) + 1.0)
    cwt = cwt * mt.astype(jnp.float32)
    dlo = jnp.concatenate([dt[:HT, i:i + 1] for i in range(TB)], axis=0)
    dhi = jnp.concatenate([dt[HT:, i:i + 1] for i in range(TB)], axis=0)
    clo = jnp.concatenate([cwt[:HT, i:i + 1] for i in range(TB)], axis=0)
    chi = jnp.concatenate([cwt[HT:, i:i + 1] for i in range(TB)], axis=0)
    out_r[...] = jnp.concatenate([dlo, dhi, clo, chi], axis=1)


def _layer_kernel(ti_r, tj_r, cnt_r,
                  hi_r, hj_r, dc_r,
                  cw1_r, mw1_r, mb1_r, mw2_r, mb2_r,
                  cw2_r, cb2_r, lw_r, lb_r,
                  out_r, acc_r):
    p = pl.program_id(0)
    cnt = cnt_r[0]
    ti = ti_r[p]
    tj = tj_r[p]
    prev_tj = tj_r[jnp.maximum(p - 1, 0)]
    next_tj = tj_r[jnp.minimum(p + 1, GRID - 1)]
    active = p < cnt
    first = jnp.logical_or(p == 0, prev_tj != tj)
    last = jnp.logical_or(p == cnt - 1, next_tj != tj)

    @pl.when(active)
    def _tile():
        xf = jnp.dot(hi_r[...], cw1_r[...],
                     preferred_element_type=jnp.float32)
        ilane = jax.lax.broadcasted_iota(jnp.int32, (1, NGP), 1)
        offs = jnp.where(ilane < NG, ilane.astype(jnp.float32) * DELTA, 0.0)

        dsl2 = jnp.concatenate(
            [jnp.broadcast_to(dc_r[:, 0:1], (P2H, NGP)),
             jnp.broadcast_to(dc_r[:, 1:2], (P2H, NGP))], axis=1)
        offs2 = jnp.concatenate([offs, offs], axis=1)
        ea = jnp.exp(COEFF * (dsl2 - offs2) ** 2)
        hid = _ssp(jnp.dot(ea, mw1_r[...],
                           preferred_element_type=jnp.float32) + mb1_r[...])
        wf = jnp.dot(hid, mw2_r[...],
                     preferred_element_type=jnp.float32) + mb2_r[...]
        cc2 = jnp.concatenate(
            [jnp.broadcast_to(dc_r[:, 2:3], (P2H, F)),
             jnp.broadcast_to(dc_r[:, 3:4], (P2H, F))], axis=1)
        msg = wf * cc2
        msg3 = msg.reshape(TB, HT, 2 * F)
        xfc = jnp.concatenate([xf, xf], axis=1)[:, None, :]
        red = jnp.sum(msg3 * xfc, axis=0)
        partial = jnp.concatenate([red[:, 0:F], red[:, F:2 * F]], axis=0)

        acc_r[...] = jnp.where(first, 0.0, acc_r[...]) + partial

    @pl.when(jnp.logical_and(active, last))
    def _finish():
        v = jnp.dot(acc_r[...], cw2_r[...],
                    preferred_element_type=jnp.float32) + cb2_r[...]
        v = _ssp(v)
        v = jnp.dot(v, lw_r[...], preferred_element_type=jnp.float32) + lb_r[...]
        out_r[...] = hj_r[...] + v


def _head_kernel(h_r, bat_r, w1_r, b1_r, w2_r, b2_r, rw_r, rb_r, out_r):
    hh = _ssp(jnp.dot(h_r[...], w1_r[...],
                      preferred_element_type=jnp.float32) + b1_r[...])
    hh = jnp.dot(hh, w2_r[...], preferred_element_type=jnp.float32) + b2_r[...]
    oh = (jax.lax.broadcasted_iota(jnp.int32, (B, N), 0) == bat_r[...])
    g = jnp.dot(oh.astype(jnp.float32), hh, preferred_element_type=jnp.float32)
    out_r[...] = jnp.dot(g, rw_r[...], preferred_element_type=jnp.float32) + rb_r[...]


def kernel(z, pos, batch, emb_table, mlp_w1, mlp_b1, mlp_w2, mlp_b2,
           conv_w1, conv_w2, conv_b2, lin_w, lin_b,
           head1_w, head1_b, head2_w, head2_b, reg_w, reg_b):
    z = z.astype(jnp.int32)
    batch = batch.astype(jnp.int32)

    bmat = batch.reshape(NB, TB)
    bmin, bmax = bmat[:, 0], bmat[:, -1]
    ov = (bmin[:, None] <= bmax[None, :]) & (bmin[None, :] <= bmax[:, None])
    flat = ov.T.reshape(-1)
    idx = jnp.nonzero(flat, size=GRID, fill_value=0)[0]
    cnt = flat.sum().astype(jnp.int32)
    last_idx = idx[cnt - 1]
    idx = jnp.where(jnp.arange(GRID) < cnt, idx, last_idx).astype(jnp.int32)
    tj_arr = (idx // NB).astype(jnp.int32)
    ti_arr = (idx % NB).astype(jnp.int32)
    cnt_arr = cnt.reshape(1)

    emb_p = jnp.zeros((ZPAD, H), jnp.float32).at[:100].set(emb_table)
    h = pl.pallas_call(
        _embed_kernel,
        grid=(NB,),
        in_specs=[pl.BlockSpec((TB, 1), lambda i: (i, 0)),
                  pl.BlockSpec((ZPAD, H), lambda i: (0, 0))],
        out_specs=pl.BlockSpec((TB, H), lambda i: (i, 0)),
        out_shape=jax.ShapeDtypeStruct((N, H), jnp.float32),
    )(z.reshape(N, 1), emb_p)

    post = jnp.zeros((NB, 8, TB), jnp.float32).at[:, :3, :].set(
        pos.reshape(NB, TB, 3).transpose(0, 2, 1))
    bcol = batch.reshape(N, 1)
    brow = batch.reshape(NB, 1, TB)
    mlp_w1p = jnp.zeros((L, NGP, F), jnp.float32).at[:, :NG].set(mlp_w1)
    w1d = jnp.zeros((L, 2 * NGP, 2 * F), jnp.float32)
    w1d = w1d.at[:, :NGP, :F].set(mlp_w1p).at[:, NGP:, F:].set(mlp_w1p)
    w2d = jnp.zeros((L, 2 * F, 2 * F), jnp.float32)
    w2d = w2d.at[:, :F, :F].set(mlp_w2).at[:, F:, F:].set(mlp_w2)
    b1d = jnp.concatenate([mlp_b1, mlp_b1], axis=1).reshape(L, 1, 2 * F)
    b2d = jnp.concatenate([mlp_b2, mlp_b2], axis=1).reshape(L, 1, 2 * F)

    dc = pl.pallas_call(
        _pair_kernel,
        grid_spec=pltpu.PrefetchScalarGridSpec(
            num_scalar_prefetch=3,
            grid=(cnt,),
            in_specs=[
                pl.BlockSpec((TB, 3), lambda p, ti, tj, c: (tj[p], 0)),
                pl.BlockSpec((1, 8, TB), lambda p, ti, tj, c: (ti[p], 0, 0)),
                pl.BlockSpec((TB, 1), lambda p, ti, tj, c: (tj[p], 0)),
                pl.BlockSpec((1, 1, TB), lambda p, ti, tj, c: (ti[p], 0, 0)),
            ],
            out_specs=pl.BlockSpec((P2H, 4), lambda p, ti, tj, c: (p, 0)),
        ),
        out_shape=jax.ShapeDtypeStruct((GRID * P2H, 4), jnp.float32),
    )(ti_arr, tj_arr, cnt_arr, pos, post, bcol, brow)

    layer_call = pl.pallas_call(
        _layer_kernel,
        grid_spec=pltpu.PrefetchScalarGridSpec(
            num_scalar_prefetch=3,
            grid=(cnt,),
            in_specs=[
                pl.BlockSpec((TB, H), lambda p, ti, tj, c: (ti[p], 0)),
                pl.BlockSpec((TB, H), lambda p, ti, tj, c: (tj[p], 0)),
                pl.BlockSpec((P2H, 4), lambda p, ti, tj, c: (p, 0)),
                pl.BlockSpec((H, F), lambda p, ti, tj, c: (0, 0)),
                pl.BlockSpec((2 * NGP, 2 * F), lambda p, ti, tj, c: (0, 0)),
                pl.BlockSpec((1, 2 * F), lambda p, ti, tj, c: (0, 0)),
                pl.BlockSpec((2 * F, 2 * F), lambda p, ti, tj, c: (0, 0)),
                pl.BlockSpec((1, 2 * F), lambda p, ti, tj, c: (0, 0)),
                pl.BlockSpec((F, H), lambda p, ti, tj, c: (0, 0)),
                pl.BlockSpec((1, H), lambda p, ti, tj, c: (0, 0)),
                pl.BlockSpec((H, H), lambda p, ti, tj, c: (0, 0)),
                pl.BlockSpec((1, H), lambda p, ti, tj, c: (0, 0)),
            ],
            out_specs=pl.BlockSpec((TB, H), lambda p, ti, tj, c: (tj[p], 0)),
            scratch_shapes=[pltpu.VMEM((TB, F), jnp.float32)],
        ),
        out_shape=jax.ShapeDtypeStruct((N, H), jnp.float32),
    )
    for l in range(L):
        h = layer_call(ti_arr, tj_arr, cnt_arr, h, h, dc,
                       conv_w1[l], w1d[l], b1d[l],
                       w2d[l], b2d[l],
                       conv_w2[l], conv_b2[l].reshape(1, H),
                       lin_w[l], lin_b[l].reshape(1, H))

    out = pl.pallas_call(
        _head_kernel,
        grid=(1,),
        in_specs=[
            pl.BlockSpec((N, H), lambda i: (0, 0)),
            pl.BlockSpec((1, N), lambda i: (0, 0)),
            pl.BlockSpec((H, H // 2), lambda i: (0, 0)),
            pl.BlockSpec((1, H // 2), lambda i: (0, 0)),
            pl.BlockSpec((H // 2, H), lambda i: (0, 0)),
            pl.BlockSpec((1, H), lambda i: (0, 0)),
            pl.BlockSpec((H, 1), lambda i: (0, 0)),
            pl.BlockSpec((1, 1), lambda i: (0, 0)),
        ],
        out_specs=pl.BlockSpec((B, 1), lambda i: (0, 0)),
        out_shape=jax.ShapeDtypeStruct((B, 1), jnp.float32),
    )(h, batch.reshape(1, N), head1_w, head1_b.reshape(1, H // 2),
      head2_w, head2_b.reshape(1, H), reg_w, reg_b.reshape(1, 1))
    return out


# speedup vs baseline: 1.6696x; 1.6696x over previous
import math

import jax
import jax.numpy as jnp
from jax.experimental import pallas as pl
from jax.experimental.pallas import tpu as pltpu

H = 128
F = 128
L = 6
NG = 50
NGP = 64
CUTOFF = 10.0
N = 4096
B = 64
TB = 64
NB = N // TB
GRID = NB * NB
SI = 64
ZPAD = 128
HT = TB // 2
P2H = TB * TB // 2
LOG2 = math.log(2.0)
LOG2E = 1.0 / math.log(2.0)
DELTA = CUTOFF / (NG - 1)
COEFF = -0.5 / DELTA ** 2


def _ssp(x):
    return jax.nn.softplus(x) - LOG2


def _embed_kernel(z_r, emb_r, out_r):
    zcol = z_r[...]
    oh = (zcol == jax.lax.broadcasted_iota(jnp.int32, (TB, ZPAD), 1))
    out_r[...] = jnp.dot(oh.astype(jnp.float32), emb_r[...],
                         preferred_element_type=jnp.float32)


def _pair_kernel(ti_r, tj_r, cnt_r, posj_r, posit_r, bj_r, bi_r, out_r):
    p = pl.program_id(0)
    ti = ti_r[p]
    tj = tj_r[p]
    pj = posj_r[...]
    pit = posit_r[0]
    d2t = (pj[:, 0:1] - pit[0:1, :]) ** 2
    d2t = d2t + (pj[:, 1:2] - pit[1:2, :]) ** 2
    d2t = d2t + (pj[:, 2:3] - pit[2:3, :]) ** 2
    dt = jnp.sqrt(d2t)
    gj = tj * TB + jax.lax.broadcasted_iota(jnp.int32, (TB, TB), 0)
    gi = ti * TB + jax.lax.broadcasted_iota(jnp.int32, (TB, TB), 1)
    mt = (bj_r[...] == bi_r[0]) & (gj != gi) & (d2t < CUTOFF ** 2)
    cwt = 0.5 * (jnp.cos(dt * (jnp.pi / CUTOFF)) + 1.0)
    cwt = cwt * mt.astype(jnp.float32)
    dcol = jnp.concatenate([dt[:, i:i + 1] for i in range(TB)], axis=0)
    ccol = jnp.concatenate([cwt[:, i:i + 1] for i in range(TB)], axis=0)
    out_r[...] = jnp.concatenate([dcol, ccol], axis=1)


def _layer_kernel(ta_r, tb_r, cnt_r,
                  ha_r, hb_r, hfull_r, dc_r,
                  cw1_r, mw1_r, mb1_r, mw2_r, mb2_r,
                  cw2_r, cb2_r, lw_r, lb_r,
                  out_r, acc_r):
    p = pl.program_id(0)
    cnt = cnt_r[0]
    ta = ta_r[p]
    tb = tb_r[p]

    @pl.when(p == 0)
    def _init():
        acc_r[...] = jnp.zeros_like(acc_r)

    xfa = jnp.dot(ha_r[...], cw1_r[...],
                  preferred_element_type=jnp.float32)
    ilane = jax.lax.broadcasted_iota(jnp.int32, (1, NGP), 1)
    offs = jnp.where(ilane < NG, ilane.astype(jnp.float32) * DELTA, 0.0)

    dsl = dc_r[:, 0:1]
    ea = jnp.exp(COEFF * (dsl - offs) ** 2)
    hid = _ssp(jnp.dot(ea, mw1_r[...],
                       preferred_element_type=jnp.float32) + mb1_r[...])
    wf = jnp.dot(hid, mw2_r[...],
                 preferred_element_type=jnp.float32) + mb2_r[...]
    wf = wf * dc_r[:, 1:2]
    wf3 = wf.reshape(TB, TB, F)
    partial_b = jnp.sum(wf3 * xfa[:, None, :], axis=0)
    acc_r[pl.ds(tb * TB, TB), :] += partial_b

    @pl.when(ta != tb)
    def _sym():
        xfb = jnp.dot(hb_r[...], cw1_r[...],
                      preferred_element_type=jnp.float32)
        partial_a = jnp.sum(wf3 * xfb[None, :, :], axis=1)
        acc_r[pl.ds(ta * TB, TB), :] += partial_a

    @pl.when(p == cnt - 1)
    def _finish():
        v = jnp.dot(acc_r[...], cw2_r[...],
                    preferred_element_type=jnp.float32) + cb2_r[...]
        v = _ssp(v)
        v = jnp.dot(v, lw_r[...], preferred_element_type=jnp.float32) + lb_r[...]
        out_r[...] = hfull_r[...] + v


def _head_kernel(h_r, bat_r, w1_r, b1_r, w2_r, b2_r, rw_r, rb_r, out_r):
    hh = _ssp(jnp.dot(h_r[...], w1_r[...],
                      preferred_element_type=jnp.float32) + b1_r[...])
    hh = jnp.dot(hh, w2_r[...], preferred_element_type=jnp.float32) + b2_r[...]
    oh = (jax.lax.broadcasted_iota(jnp.int32, (B, N), 0) == bat_r[...])
    g = jnp.dot(oh.astype(jnp.float32), hh, preferred_element_type=jnp.float32)
    out_r[...] = jnp.dot(g, rw_r[...], preferred_element_type=jnp.float32) + rb_r[...]


def kernel(z, pos, batch, emb_table, mlp_w1, mlp_b1, mlp_w2, mlp_b2,
           conv_w1, conv_w2, conv_b2, lin_w, lin_b,
           head1_w, head1_b, head2_w, head2_b, reg_w, reg_b):
    z = z.astype(jnp.int32)
    batch = batch.astype(jnp.int32)

    bmat = batch.reshape(NB, TB)
    bmin, bmax = bmat[:, 0], bmat[:, -1]
    ov = (bmin[:, None] <= bmax[None, :]) & (bmin[None, :] <= bmax[:, None])
    ii = jnp.arange(NB)
    ov = ov & (ii[:, None] <= ii[None, :])
    flat = ov.reshape(-1)
    idx = jnp.nonzero(flat, size=GRID, fill_value=0)[0]
    cnt = flat.sum().astype(jnp.int32)
    last_idx = idx[cnt - 1]
    idx = jnp.where(jnp.arange(GRID) < cnt, idx, last_idx).astype(jnp.int32)
    ta_arr = (idx // NB).astype(jnp.int32)
    tb_arr = (idx % NB).astype(jnp.int32)
    cnt_arr = cnt.reshape(1)

    emb_p = jnp.zeros((ZPAD, H), jnp.float32).at[:100].set(emb_table)
    h = pl.pallas_call(
        _embed_kernel,
        grid=(NB,),
        in_specs=[pl.BlockSpec((TB, 1), lambda i: (i, 0)),
                  pl.BlockSpec((ZPAD, H), lambda i: (0, 0))],
        out_specs=pl.BlockSpec((TB, H), lambda i: (i, 0)),
        out_shape=jax.ShapeDtypeStruct((N, H), jnp.float32),
    )(z.reshape(N, 1), emb_p)

    post = jnp.zeros((NB, 8, TB), jnp.float32).at[:, :3, :].set(
        pos.reshape(NB, TB, 3).transpose(0, 2, 1))
    bcol = batch.reshape(N, 1)
    brow = batch.reshape(NB, 1, TB)
    mlp_w1p = jnp.zeros((L, NGP, F), jnp.float32).at[:, :NG].set(mlp_w1)

    dc = pl.pallas_call(
        _pair_kernel,
        grid_spec=pltpu.PrefetchScalarGridSpec(
            num_scalar_prefetch=3,
            grid=(cnt,),
            in_specs=[
                pl.BlockSpec((TB, 3), lambda p, ti, tj, c: (tj[p], 0)),
                pl.BlockSpec((1, 8, TB), lambda p, ti, tj, c: (ti[p], 0, 0)),
                pl.BlockSpec((TB, 1), lambda p, ti, tj, c: (tj[p], 0)),
                pl.BlockSpec((1, 1, TB), lambda p, ti, tj, c: (ti[p], 0, 0)),
            ],
            out_specs=pl.BlockSpec((TB * TB, 2), lambda p, ti, tj, c: (p, 0)),
        ),
        out_shape=jax.ShapeDtypeStruct((GRID * TB * TB, 2), jnp.float32),
    )(ta_arr, tb_arr, cnt_arr, pos, post, bcol, brow)

    layer_call = pl.pallas_call(
        _layer_kernel,
        grid_spec=pltpu.PrefetchScalarGridSpec(
            num_scalar_prefetch=3,
            grid=(cnt,),
            in_specs=[
                pl.BlockSpec((TB, H), lambda p, ti, tj, c: (ti[p], 0)),
                pl.BlockSpec((TB, H), lambda p, ti, tj, c: (tj[p], 0)),
                pl.BlockSpec((N, H), lambda p, ti, tj, c: (0, 0)),
                pl.BlockSpec((TB * TB, 2), lambda p, ti, tj, c: (p, 0)),
                pl.BlockSpec((H, F), lambda p, ti, tj, c: (0, 0)),
                pl.BlockSpec((NGP, F), lambda p, ti, tj, c: (0, 0)),
                pl.BlockSpec((1, F), lambda p, ti, tj, c: (0, 0)),
                pl.BlockSpec((F, F), lambda p, ti, tj, c: (0, 0)),
                pl.BlockSpec((1, F), lambda p, ti, tj, c: (0, 0)),
                pl.BlockSpec((F, H), lambda p, ti, tj, c: (0, 0)),
                pl.BlockSpec((1, H), lambda p, ti, tj, c: (0, 0)),
                pl.BlockSpec((H, H), lambda p, ti, tj, c: (0, 0)),
                pl.BlockSpec((1, H), lambda p, ti, tj, c: (0, 0)),
            ],
            out_specs=pl.BlockSpec((N, H), lambda p, ti, tj, c: (0, 0)),
            scratch_shapes=[pltpu.VMEM((N, F), jnp.float32)],
        ),
        out_shape=jax.ShapeDtypeStruct((N, H), jnp.float32),
    )
    for l in range(L):
        h = layer_call(ta_arr, tb_arr, cnt_arr, h, h, h, dc,
                       conv_w1[l], mlp_w1p[l], mlp_b1[l].reshape(1, F),
                       mlp_w2[l], mlp_b2[l].reshape(1, F),
                       conv_w2[l], conv_b2[l].reshape(1, H),
                       lin_w[l], lin_b[l].reshape(1, H))

    out = pl.pallas_call(
        _head_kernel,
        grid=(1,),
        in_specs=[
            pl.BlockSpec((N, H), lambda i: (0, 0)),
            pl.BlockSpec((1, N), lambda i: (0, 0)),
            pl.BlockSpec((H, H // 2), lambda i: (0, 0)),
            pl.BlockSpec((1, H // 2), lambda i: (0, 0)),
            pl.BlockSpec((H // 2, H), lambda i: (0, 0)),
            pl.BlockSpec((1, H), lambda i: (0, 0)),
            pl.BlockSpec((H, 1), lambda i: (0, 0)),
            pl.BlockSpec((1, 1), lambda i: (0, 0)),
        ],
        out_specs=pl.BlockSpec((B, 1), lambda i: (0, 0)),
        out_shape=jax.ShapeDtypeStruct((B, 1), jnp.float32),
    )(h, batch.reshape(1, N), head1_w, head1_b.reshape(1, H // 2),
      head2_w, head2_b.reshape(1, H), reg_w, reg_b.reshape(1, 1))
    return out
